# Initial kernel scaffold; baseline (speedup 1.0000x reference)
import jax
import jax.numpy as jnp
from jax.experimental import pallas as pl


def kernel(bucket_ids, values, prototypes, decoder_w):
    raise NotImplementedError("write your pallas kernel here")


# speedup vs baseline: 1.1182x; 1.1182x over previous
import functools

import jax
import jax.numpy as jnp
from jax import lax
from jax.experimental import pallas as pl
from jax.experimental.pallas import tpu as pltpu
from jax.experimental.pallas import tpu_sc as plsc

K_MAX = 100000
P_DIM = 64
MODEL_DIM = 64
N = 16384
RATE = 0.1

NC = 2
NS = 16
CHUNK = 128
PAD = 80

_MESH = dict(core_axis_name="c", subcore_axis_name="s", num_cores=NC,
             num_subcores=NS)
_SC_PARAMS = pltpu.CompilerParams(use_tc_tiling_on_sc=False)


def _copy_body(src, dst):
    dst[...] = src[...]


def _copy_protos(prototypes):
    blk = 2000
    grid = K_MAX // blk
    return pl.pallas_call(
        _copy_body,
        grid=(grid,),
        in_specs=[pl.BlockSpec((blk, P_DIM), lambda i: (i, 0))],
        out_specs=pl.BlockSpec((blk, P_DIM), lambda i: (i, 0)),
        out_shape=jax.ShapeDtypeStruct((K_MAX, P_DIM), jnp.float32),
    )(prototypes)


def _k1_body(protos, ids2d, iota2d, pgath, slot, idx_v, iot_v, row_v):
    wid = lax.axis_index("c") * NS + lax.axis_index("s")
    nrow = (N // CHUNK) // (NC * NS)
    pltpu.sync_copy(ids2d.at[pl.ds(wid * nrow, nrow)], idx_v)
    pltpu.sync_copy(iota2d.at[pl.ds(wid * nrow, nrow)], iot_v)
    for c in range(nrow):
        base = (wid * nrow + c) * CHUNK
        pltpu.sync_copy(protos.at[idx_v.at[c]], row_v)
        pltpu.sync_copy(row_v, pgath.at[pl.ds(base, CHUNK)])
        pltpu.sync_copy(iot_v.at[c], slot.at[idx_v.at[c]])


def _k1(prototypes, ids2d, iota2d):
    nrow = (N // CHUNK) // (NC * NS)
    return pl.kernel(
        _k1_body,
        out_type=(
            jax.ShapeDtypeStruct((N, P_DIM), jnp.float32),
            jax.ShapeDtypeStruct((K_MAX,), jnp.int32),
        ),
        mesh=plsc.VectorSubcoreMesh(**_MESH),
        compiler_params=_SC_PARAMS,
        scratch_types=[
            pltpu.VMEM((nrow, CHUNK), jnp.int32),
            pltpu.VMEM((nrow, CHUNK), jnp.int32),
            pltpu.VMEM((CHUNK, P_DIM), jnp.float32),
        ],
    )(prototypes, ids2d, iota2d)


def _k2_body(ids2d, slot, valpad, zb, sums, acc, idx_v, rep_v, buf_v):
    cid = lax.axis_index("c")
    w = lax.axis_index("s")
    nrow = (N // CHUNK) // NS

    @pl.when(cid == 0)
    def _zero():
        pltpu.sync_copy(zb, buf_v)
        for j in range(nrow):
            pltpu.sync_copy(buf_v, acc.at[pl.ds((w * nrow + j) * CHUNK,
                                                CHUNK)])

    plsc.subcore_barrier()

    @pl.when(cid == 0)
    def _accum():
        pltpu.sync_copy(ids2d.at[pl.ds(w * nrow, nrow)], idx_v)
        for c in range(nrow):
            pltpu.sync_copy(slot.at[idx_v.at[c]], rep_v.at[c])
            pltpu.sync_copy(valpad.at[pl.ds((w * nrow + c) * CHUNK, CHUNK)],
                            buf_v)
            pltpu.sync_copy(buf_v, acc.at[rep_v.at[c]], add=True)

    plsc.subcore_barrier()

    @pl.when(cid == 0)
    def _readback():
        for c in range(nrow):
            pltpu.sync_copy(acc.at[rep_v.at[c]], buf_v)
            pltpu.sync_copy(buf_v, sums.at[pl.ds((w * nrow + c) * CHUNK,
                                                 CHUNK)])


def _k2(ids2d, slot_tbl, valpad, zblock):
    nrow = (N // CHUNK) // NS
    return pl.kernel(
        _k2_body,
        out_type=jax.ShapeDtypeStruct((N, PAD), jnp.float32),
        mesh=plsc.VectorSubcoreMesh(**_MESH),
        compiler_params=_SC_PARAMS,
        scratch_types=[
            pltpu.VMEM_SHARED((N, PAD), jnp.float32),
            pltpu.VMEM((nrow, CHUNK), jnp.int32),
            pltpu.VMEM((nrow, CHUNK), jnp.int32),
            pltpu.VMEM((CHUNK, PAD), jnp.float32),
        ],
    )(ids2d, slot_tbl, valpad, zblock)


def _k3_body(pg, sums, w, fin, dec):
    s = sums[:, :P_DIM]
    cnt = sums[:, P_DIM:P_DIM + 1]
    f = (1.0 - RATE) * pg[...] + RATE * (s / cnt)
    fin[...] = f
    dec[...] = lax.dot_general(f, w[...], (((1,), (1,)), ((), ())),
                               preferred_element_type=jnp.float32)


def _k3(p_gath, sums_g, decoder_w):
    blk = 2048
    grid = N // blk
    return pl.pallas_call(
        _k3_body,
        grid=(grid,),
        in_specs=[
            pl.BlockSpec((blk, P_DIM), lambda i: (i, 0)),
            pl.BlockSpec((blk, PAD), lambda i: (i, 0)),
            pl.BlockSpec((MODEL_DIM, P_DIM), lambda i: (0, 0)),
        ],
        out_specs=(
            pl.BlockSpec((blk, P_DIM), lambda i: (i, 0)),
            pl.BlockSpec((blk, MODEL_DIM), lambda i: (i, 0)),
        ),
        out_shape=(
            jax.ShapeDtypeStruct((N, P_DIM), jnp.float32),
            jax.ShapeDtypeStruct((N, MODEL_DIM), jnp.float32),
        ),
    )(p_gath, sums_g, decoder_w)


def _k4_body(ids2d, fin, out_ref, idx_v, row_v):
    wid = lax.axis_index("c") * NS + lax.axis_index("s")
    nrow = (N // CHUNK) // (NC * NS)
    pltpu.sync_copy(ids2d.at[pl.ds(wid * nrow, nrow)], idx_v)
    for c in range(nrow):
        base = (wid * nrow + c) * CHUNK
        pltpu.sync_copy(fin.at[pl.ds(base, CHUNK)], row_v)
        pltpu.sync_copy(row_v, out_ref.at[idx_v.at[c]])


def _k4(ids2d, final, out_ref):
    nrow = (N // CHUNK) // (NC * NS)
    pl.kernel(
        _k4_body,
        out_type=(),
        mesh=plsc.VectorSubcoreMesh(**_MESH),
        compiler_params=_SC_PARAMS,
        scratch_types=[
            pltpu.VMEM((nrow, CHUNK), jnp.int32),
            pltpu.VMEM((CHUNK, P_DIM), jnp.float32),
        ],
    )(ids2d, final, out_ref)


def kernel(bucket_ids, values, prototypes, decoder_w):
    ids = bucket_ids.astype(jnp.int32)
    ids2d = ids.reshape(N // CHUNK, CHUNK)
    iota2d = jnp.arange(N, dtype=jnp.int32).reshape(N // CHUNK, CHUNK)
    valpad = jnp.concatenate(
        [values, jnp.ones((N, PAD - P_DIM), jnp.float32)], axis=1)
    zblock = jnp.zeros((CHUNK, PAD), jnp.float32)

    out0 = _copy_protos(prototypes)
    p_gath, slot_tbl = _k1(prototypes, ids2d, iota2d)
    sums_g = _k2(ids2d, slot_tbl, valpad, zblock)
    final, decoded = _k3(p_gath, sums_g, decoder_w)

    out_ref = jax.new_ref(out0)
    _k4(ids2d, final, out_ref)
    return out_ref[...], decoded


# speedup vs baseline: 1.5293x; 1.3676x over previous
import jax
import jax.numpy as jnp
from jax import lax
from jax.experimental import pallas as pl
from jax.experimental.pallas import tpu as pltpu
from jax.experimental.pallas import tpu_sc as plsc

K_MAX = 100000
P_DIM = 64
MODEL_DIM = 64
N = 16384
RATE = 0.1

NC = 2
NS = 16
CHUNK = 128
PAD = 80

_MESH = dict(core_axis_name="c", subcore_axis_name="s", num_cores=NC,
             num_subcores=NS)
_SC_PARAMS = pltpu.CompilerParams(use_tc_tiling_on_sc=False)


def _k1_body(protos, ids2d, iota2d, pgath, slot, idx_v, iot_v, row_v,
             gsem, ssem, wsem):
    wid = lax.axis_index("c") * NS + lax.axis_index("s")
    nrow = (N // CHUNK) // (NC * NS)
    pltpu.sync_copy(ids2d.at[pl.ds(wid * nrow, nrow)], idx_v)
    pltpu.sync_copy(iota2d.at[pl.ds(wid * nrow, nrow)], iot_v)
    gets = [pltpu.async_copy(protos.at[idx_v.at[c]], row_v.at[c], gsem)
            for c in range(nrow)]
    puts = [pltpu.async_copy(iot_v.at[c], slot.at[idx_v.at[c]], ssem)
            for c in range(nrow)]
    wrs = []
    for c in range(nrow):
        gets[c].wait()
        base = (wid * nrow + c) * CHUNK
        wrs.append(pltpu.async_copy(row_v.at[c], pgath.at[pl.ds(base, CHUNK)],
                                    wsem))
    for d in puts + wrs:
        d.wait()


def _k1(prototypes, ids2d, iota2d):
    nrow = (N // CHUNK) // (NC * NS)
    return pl.kernel(
        _k1_body,
        out_type=(
            jax.ShapeDtypeStruct((N, P_DIM), jnp.float32),
            jax.ShapeDtypeStruct((K_MAX,), jnp.int32),
        ),
        mesh=plsc.VectorSubcoreMesh(**_MESH),
        compiler_params=_SC_PARAMS,
        scratch_types=[
            pltpu.VMEM((nrow, CHUNK), jnp.int32),
            pltpu.VMEM((nrow, CHUNK), jnp.int32),
            pltpu.VMEM((nrow, CHUNK, P_DIM), jnp.float32),
            pltpu.SemaphoreType.DMA,
            pltpu.SemaphoreType.DMA,
            pltpu.SemaphoreType.DMA,
        ],
    )(prototypes, ids2d, iota2d)


_DEPTH = 3


def _k2_body(ids2d, slot, valpad, zb, sums, acc, idx_v, rep_v, buf_v,
             s1, s2, s3):
    cid = lax.axis_index("c")
    w = lax.axis_index("s")
    nrow = (N // CHUNK) // NS

    @pl.when(cid == 0)
    def _zero():
        pltpu.sync_copy(zb, buf_v.at[0])
        zs = [pltpu.async_copy(
            buf_v.at[0], acc.at[pl.ds((w * nrow + j) * CHUNK, CHUNK)], s1)
            for j in range(nrow)]
        for d in zs:
            d.wait()

    plsc.subcore_barrier()

    @pl.when(cid == 0)
    def _accum():
        pltpu.sync_copy(ids2d.at[pl.ds(w * nrow, nrow)], idx_v)
        reps = [pltpu.async_copy(slot.at[idx_v.at[c]], rep_v.at[c], s1)
                for c in range(nrow)]
        vals = {c: pltpu.async_copy(
            valpad.at[pl.ds((w * nrow + c) * CHUNK, CHUNK)],
            buf_v.at[c % _DEPTH], s2) for c in range(_DEPTH)}
        adds = {}
        for c in range(nrow):
            reps[c].wait()
            vals[c].wait()
            adds[c] = pltpu.async_copy(buf_v.at[c % _DEPTH],
                                       acc.at[rep_v.at[c]], s3, add=True)
            nc = c + _DEPTH
            if nc < nrow:
                adds[c].wait()
                vals[nc] = pltpu.async_copy(
                    valpad.at[pl.ds((w * nrow + nc) * CHUNK, CHUNK)],
                    buf_v.at[nc % _DEPTH], s2)
        for c in range(max(0, nrow - _DEPTH), nrow):
            adds[c].wait()

    plsc.subcore_barrier()

    @pl.when(cid == 0)
    def _readback():
        gets = {c: pltpu.async_copy(acc.at[rep_v.at[c]],
                                    buf_v.at[c % _DEPTH], s1)
                for c in range(_DEPTH)}
        wrs = {}
        for c in range(nrow):
            gets[c].wait()
            wrs[c] = pltpu.async_copy(
                buf_v.at[c % _DEPTH],
                sums.at[pl.ds((w * nrow + c) * CHUNK, CHUNK)], s2)
            nc = c + _DEPTH
            if nc < nrow:
                wrs[c].wait()
                gets[nc] = pltpu.async_copy(acc.at[rep_v.at[nc]],
                                            buf_v.at[nc % _DEPTH], s1)
        for c in range(max(0, nrow - _DEPTH), nrow):
            wrs[c].wait()


def _k2(ids2d, slot_tbl, valpad, zblock):
    nrow = (N // CHUNK) // NS
    return pl.kernel(
        _k2_body,
        out_type=jax.ShapeDtypeStruct((N, PAD), jnp.float32),
        mesh=plsc.VectorSubcoreMesh(**_MESH),
        compiler_params=_SC_PARAMS,
        scratch_types=[
            pltpu.VMEM_SHARED((N, PAD), jnp.float32),
            pltpu.VMEM((nrow, CHUNK), jnp.int32),
            pltpu.VMEM((nrow, CHUNK), jnp.int32),
            pltpu.VMEM((_DEPTH, CHUNK, PAD), jnp.float32),
            pltpu.SemaphoreType.DMA,
            pltpu.SemaphoreType.DMA,
            pltpu.SemaphoreType.DMA,
        ],
    )(ids2d, slot_tbl, valpad, zblock)


def _k3_body(pg, sums, w, fin, dec):
    s = sums[:, :P_DIM]
    cnt = sums[:, P_DIM:P_DIM + 1]
    f = (1.0 - RATE) * pg[...] + RATE * (s / cnt)
    fin[...] = f
    dec[...] = lax.dot_general(f, w[...], (((1,), (1,)), ((), ())),
                               preferred_element_type=jnp.float32)


def _k3(p_gath, sums_g, decoder_w):
    blk = 2048
    grid = N // blk
    return pl.pallas_call(
        _k3_body,
        grid=(grid,),
        in_specs=[
            pl.BlockSpec((blk, P_DIM), lambda i: (i, 0)),
            pl.BlockSpec((blk, PAD), lambda i: (i, 0)),
            pl.BlockSpec((MODEL_DIM, P_DIM), lambda i: (0, 0)),
        ],
        out_specs=(
            pl.BlockSpec((blk, P_DIM), lambda i: (i, 0)),
            pl.BlockSpec((blk, MODEL_DIM), lambda i: (i, 0)),
        ),
        out_shape=(
            jax.ShapeDtypeStruct((N, P_DIM), jnp.float32),
            jax.ShapeDtypeStruct((N, MODEL_DIM), jnp.float32),
        ),
    )(p_gath, sums_g, decoder_w)


def _k4_body(ids2d, fin, out_ref, idx_v, row_v, gsem, ssem):
    wid = lax.axis_index("c") * NS + lax.axis_index("s")
    nrow = (N // CHUNK) // (NC * NS)
    pltpu.sync_copy(ids2d.at[pl.ds(wid * nrow, nrow)], idx_v)
    gets = [pltpu.async_copy(
        fin.at[pl.ds((wid * nrow + c) * CHUNK, CHUNK)], row_v.at[c], gsem)
        for c in range(nrow)]
    puts = []
    for c in range(nrow):
        gets[c].wait()
        puts.append(pltpu.async_copy(row_v.at[c], out_ref.at[idx_v.at[c]],
                                     ssem))
    for d in puts:
        d.wait()


def _k4(ids2d, final, out_ref):
    nrow = (N // CHUNK) // (NC * NS)
    pl.kernel(
        _k4_body,
        out_type=(),
        mesh=plsc.VectorSubcoreMesh(**_MESH),
        compiler_params=_SC_PARAMS,
        scratch_types=[
            pltpu.VMEM((nrow, CHUNK), jnp.int32),
            pltpu.VMEM((nrow, CHUNK, P_DIM), jnp.float32),
            pltpu.SemaphoreType.DMA,
            pltpu.SemaphoreType.DMA,
        ],
    )(ids2d, final, out_ref)


def kernel(bucket_ids, values, prototypes, decoder_w):
    ids = bucket_ids.astype(jnp.int32)
    ids2d = ids.reshape(N // CHUNK, CHUNK)
    iota2d = jnp.arange(N, dtype=jnp.int32).reshape(N // CHUNK, CHUNK)
    valpad = jnp.concatenate(
        [values, jnp.ones((N, PAD - P_DIM), jnp.float32)], axis=1)
    zblock = jnp.zeros((CHUNK, PAD), jnp.float32)

    p_gath, slot_tbl = _k1(prototypes, ids2d, iota2d)
    sums_g = _k2(ids2d, slot_tbl, valpad, zblock)
    final, decoded = _k3(p_gath, sums_g, decoder_w)

    out_ref = jax.new_ref(prototypes)
    _k4(ids2d, final, out_ref)
    return out_ref[...], decoded
